# Initial kernel scaffold; baseline (speedup 1.0000x reference)
import jax
import jax.numpy as jnp
from jax.experimental import pallas as pl


def kernel(layer_in, weight, delay):
    raise NotImplementedError("write your pallas kernel here")


# speedup vs baseline: 3.2483x; 3.2483x over previous
import functools

import jax
import jax.numpy as jnp
from jax.experimental import pallas as pl

MAX_SPIKE_TIME = 100000.0


def _snn_body(x_ref, w_ref, d_ref, o_ref, *, n):
    x = x_ref[...]
    d = d_ref[...]
    bb = x.shape[0]
    x = x * jnp.exp(jnp.maximum(d, 0.0))
    xs = jnp.concatenate([x, jnp.ones((bb, 1), jnp.float32)], axis=1)
    w = w_ref[...]
    lane = jax.lax.broadcasted_iota(jnp.int32, (1, n), 1)

    def body(k, acc):
        t = jnp.max(jnp.where(lane == k, xs, -jnp.inf), axis=1, keepdims=True)
        le = (xs < t) | ((xs == t) & (lane <= k))
        m = le.astype(jnp.float32)
        dn = (((1,), (0,)), ((), ()))
        sw = jax.lax.dot_general(m, w, dn, precision=jax.lax.Precision.HIGHEST,
                                 preferred_element_type=jnp.float32)
        swx = jax.lax.dot_general(m * xs, w, dn, precision=jax.lax.Precision.HIGHEST,
                                  preferred_element_type=jnp.float32)
        nxt = jnp.min(jnp.where(le, MAX_SPIKE_TIME, xs), axis=1, keepdims=True)
        den = jnp.maximum(sw - 1.0, 1e-10)
        cand = swx / den
        cand = jnp.where(sw < 1.0, MAX_SPIKE_TIME, cand)
        cand = jnp.where(cand < t, MAX_SPIKE_TIME, cand)
        cand = jnp.where(cand > nxt, MAX_SPIKE_TIME, cand)
        return jnp.minimum(acc, cand)

    acc0 = jnp.full((bb, w.shape[1]), MAX_SPIKE_TIME, jnp.float32)
    o_ref[...] = jax.lax.fori_loop(0, n, body, acc0)


def kernel(layer_in, weight, delay, *, block_b=512, interpret=False):
    b, in_size = layer_in.shape
    out_size = weight.shape[1]
    n = in_size + 1
    bb = block_b if b % block_b == 0 else b
    d2 = delay.reshape(1, in_size)
    return pl.pallas_call(
        functools.partial(_snn_body, n=n),
        grid=(b // bb,),
        in_specs=[
            pl.BlockSpec((bb, in_size), lambda i: (i, 0)),
            pl.BlockSpec((n, out_size), lambda i: (0, 0)),
            pl.BlockSpec((1, in_size), lambda i: (0, 0)),
        ],
        out_specs=pl.BlockSpec((bb, out_size), lambda i: (i, 0)),
        out_shape=jax.ShapeDtypeStruct((b, out_size), jnp.float32),
        interpret=interpret,
    )(layer_in, weight, d2)


# speedup vs baseline: 7.6504x; 2.3552x over previous
import functools

import jax
import jax.numpy as jnp
from jax import lax
from jax.experimental import pallas as pl
from jax.experimental.pallas import tpu as pltpu
from jax.experimental.pallas import tpu_sc as plsc

MAX_SPIKE_TIME = 100000.0
_INF = float("inf")


def _minmax(ak, av, bk, bv):
    m = ak <= bk
    return (jnp.where(m, ak, bk), jnp.where(m, av, bv),
            jnp.where(m, bk, ak), jnp.where(m, bv, av))


def _rev(x):
    return lax.rev(x, (0,))


def _merge16x2(ak, av, bk, bv):
    lok, lov, hik, hiv = _minmax(ak, av, _rev(bk), _rev(bv))
    return plsc.sort_key_val(lok, lov) + plsc.sort_key_val(hik, hiv)


def _merge32x2(a, b):
    ak0, av0, ak1, av1 = a
    bk0, bv0, bk1, bv1 = b
    l0k, l0v, h0k, h0v = _minmax(ak0, av0, _rev(bk1), _rev(bv1))
    l1k, l1v, h1k, h1v = _minmax(ak1, av1, _rev(bk0), _rev(bv0))
    a0k, a0v, a1k, a1v = _minmax(l0k, l0v, l1k, l1v)
    b0k, b0v, b1k, b1v = _minmax(h0k, h0v, h1k, h1v)
    return (plsc.sort_key_val(a0k, a0v) + plsc.sort_key_val(a1k, a1v),
            plsc.sort_key_val(b0k, b0v) + plsc.sort_key_val(b1k, b1v))


def _merge64x16(c, dk, dv):
    k0, v0, k1, v1, k2, v2, k3, v3 = c
    m3k, m3v, x3k, x3v = _minmax(k3, v3, _rev(dk), _rev(dv))
    l0k, l0v, h0k, h0v = _minmax(k0, v0, k2, v2)
    l1k, l1v, h1k, h1v = _minmax(k1, v1, m3k, m3v)
    a0k, a0v, a1k, a1v = _minmax(l0k, l0v, l1k, l1v)
    b0k, b0v, b1k, b1v = _minmax(h0k, h0v, h1k, h1v)
    return (plsc.sort_key_val(a0k, a0v) + plsc.sort_key_val(a1k, a1v) +
            plsc.sort_key_val(b0k, b0v) + plsc.sort_key_val(b1k, b1v) +
            plsc.sort_key_val(x3k, x3v))


def _sc_body(x_hbm, w_hbm, d_hbm, o_hbm, x_v, w_v, d_v, srt_k, srt_i, out_v,
             *, rows_per, nc):
    wid = lax.axis_index("s") * nc + lax.axis_index("c")
    base = wid * rows_per
    pltpu.sync_copy(x_hbm.at[pl.ds(base, rows_per)], x_v)
    pltpu.sync_copy(w_hbm, w_v)
    pltpu.sync_copy(d_hbm, d_v)

    iota = lax.iota(jnp.int32, 16)
    scale = [jnp.exp(jnp.maximum(d_v[pl.ds(16 * c, 16)], 0.0))
             for c in range(4)]
    ids = [iota + 16 * c for c in range(5)]
    k4 = jnp.where(iota == 0, jnp.float32(1.0), jnp.float32(_INF))

    def row_body(r, _):
        keys = [x_v[r, pl.ds(16 * c, 16)] * scale[c] for c in range(4)] + [k4]
        s = [plsc.sort_key_val(k, v) for k, v in zip(keys, ids)]
        a = _merge16x2(*s[0], *s[1])
        b = _merge16x2(*s[2], *s[3])
        c64a, c64b = _merge32x2(a, b)
        full = _merge64x16(c64a + c64b, *s[4])
        for g in range(5):
            srt_k[pl.ds(16 * g, 16)] = full[2 * g]
            srt_i[pl.ds(16 * g, 16)] = full[2 * g + 1]

        zero = jnp.zeros((16,), jnp.float32)
        big = jnp.full((16,), MAX_SPIKE_TIME, jnp.float32)
        one = jnp.ones((16,), jnp.float32)
        carry0 = (zero, zero, zero, zero, zero, zero, zero, zero,
                  big, big, big, big, one, one, one, one)

        def steps(carry, kg, kg1, ig, nsub):
            ws = list(carry[0:4])
            wis = list(carry[4:8])
            bnum = list(carry[8:12])
            bden = list(carry[12:16])
            for sub in range(nsub):
                t = kg[sub]
                nxt = kg1[sub]
                idx = ig[sub]
                for c in range(4):
                    wv = w_v[idx, pl.ds(16 * c, 16)]
                    ws[c] = ws[c] + wv
                    wis[c] = wis[c] + wv * t
                    den0 = ws[c] - 1.0
                    den = jnp.maximum(den0, 1e-10)
                    u = ((wis[c] >= t * den) & (wis[c] <= nxt * den)
                         & (den0 >= 0.0) & (wis[c] * bden[c] < bnum[c] * den))
                    bnum[c] = jnp.where(u, wis[c], bnum[c])
                    bden[c] = jnp.where(u, den, bden[c])
            return tuple(ws) + tuple(wis) + tuple(bnum) + tuple(bden)

        def group(g, carry):
            kg = srt_k[pl.ds(16 * g, 16)]
            kg1 = srt_k[pl.ds(16 * g + 1, 16)]
            ig = srt_i[pl.ds(16 * g, 16)]
            return steps(carry, kg, kg1, ig, 16)

        res = lax.fori_loop(0, 4, group, carry0)
        res = steps(res, srt_k[pl.ds(64, 16)], srt_k[pl.ds(65, 16)],
                    srt_i[pl.ds(64, 16)], 1)
        for c in range(4):
            out_v[r, pl.ds(16 * c, 16)] = res[8 + c] / res[12 + c]
        return 0

    lax.fori_loop(0, rows_per, row_body, 0)
    pltpu.sync_copy(out_v, o_hbm.at[pl.ds(base, rows_per)])


def kernel(layer_in, weight, delay):
    b, in_size = layer_in.shape
    n, out_size = weight.shape
    info = plsc.get_sparse_core_info()
    nc, ns = info.num_cores, info.num_subcores
    nw = nc * ns
    assert b % nw == 0
    rows_per = b // nw
    mesh = plsc.VectorSubcoreMesh(core_axis_name="c", subcore_axis_name="s")
    f = pl.kernel(
        functools.partial(_sc_body, rows_per=rows_per, nc=nc),
        mesh=mesh,
        compiler_params=pltpu.CompilerParams(needs_layout_passes=False),
        out_type=jax.ShapeDtypeStruct((b, out_size), jnp.float32),
        scratch_types=[
            pltpu.VMEM((rows_per, in_size), jnp.float32),
            pltpu.VMEM((n, out_size), jnp.float32),
            pltpu.VMEM((in_size,), jnp.float32),
            pltpu.VMEM((96,), jnp.float32),
            pltpu.VMEM((96,), jnp.int32),
            pltpu.VMEM((rows_per, out_size), jnp.float32),
        ],
    )
    return f(layer_in, weight, delay)


# speedup vs baseline: 8.5993x; 1.1240x over previous
import functools

import jax
import jax.numpy as jnp
from jax import lax
from jax.experimental import pallas as pl
from jax.experimental.pallas import tpu as pltpu
from jax.experimental.pallas import tpu_sc as plsc

MAX_SPIKE_TIME = 100000.0
_INF = float("inf")


def _minmax(ak, av, bk, bv):
    m = ak <= bk
    return (jnp.where(m, ak, bk), jnp.where(m, av, bv),
            jnp.where(m, bk, ak), jnp.where(m, bv, av))


def _rev(x):
    return lax.rev(x, (0,))


def _merge16x2(ak, av, bk, bv):
    lok, lov, hik, hiv = _minmax(ak, av, _rev(bk), _rev(bv))
    return plsc.sort_key_val(lok, lov) + plsc.sort_key_val(hik, hiv)


def _merge32x2(a, b):
    ak0, av0, ak1, av1 = a
    bk0, bv0, bk1, bv1 = b
    l0k, l0v, h0k, h0v = _minmax(ak0, av0, _rev(bk1), _rev(bv1))
    l1k, l1v, h1k, h1v = _minmax(ak1, av1, _rev(bk0), _rev(bv0))
    a0k, a0v, a1k, a1v = _minmax(l0k, l0v, l1k, l1v)
    b0k, b0v, b1k, b1v = _minmax(h0k, h0v, h1k, h1v)
    return (plsc.sort_key_val(a0k, a0v) + plsc.sort_key_val(a1k, a1v),
            plsc.sort_key_val(b0k, b0v) + plsc.sort_key_val(b1k, b1v))


def _merge64x16(c, dk, dv):
    k0, v0, k1, v1, k2, v2, k3, v3 = c
    m3k, m3v, x3k, x3v = _minmax(k3, v3, _rev(dk), _rev(dv))
    l0k, l0v, h0k, h0v = _minmax(k0, v0, k2, v2)
    l1k, l1v, h1k, h1v = _minmax(k1, v1, m3k, m3v)
    a0k, a0v, a1k, a1v = _minmax(l0k, l0v, l1k, l1v)
    b0k, b0v, b1k, b1v = _minmax(h0k, h0v, h1k, h1v)
    return (plsc.sort_key_val(a0k, a0v) + plsc.sort_key_val(a1k, a1v) +
            plsc.sort_key_val(b0k, b0v) + plsc.sort_key_val(b1k, b1v) +
            plsc.sort_key_val(x3k, x3v))


def _sc_body(x_hbm, w_hbm, d_hbm, o_hbm, x_v, w_v, d_v, srt_k, srt_i, out_v,
             *, rows_per, nc):
    wid = lax.axis_index("s") * nc + lax.axis_index("c")
    base = wid * rows_per
    pltpu.sync_copy(x_hbm.at[pl.ds(base, rows_per)], x_v)
    pltpu.sync_copy(w_hbm, w_v)
    pltpu.sync_copy(d_hbm, d_v)

    iota = lax.iota(jnp.int32, 16)
    scale = [jnp.exp(jnp.maximum(d_v[pl.ds(16 * c, 16)], 0.0))
             for c in range(4)]
    ids = [iota + 16 * c for c in range(5)]
    k4 = jnp.where(iota == 0, jnp.float32(1.0), jnp.float32(_INF))

    def row_body(r, _):
        keys = [x_v[r, pl.ds(16 * c, 16)] * scale[c] for c in range(4)] + [k4]
        s = [plsc.sort_key_val(k, v) for k, v in zip(keys, ids)]
        a = _merge16x2(*s[0], *s[1])
        b = _merge16x2(*s[2], *s[3])
        c64a, c64b = _merge32x2(a, b)
        full = _merge64x16(c64a + c64b, *s[4])
        for g in range(5):
            srt_k[pl.ds(16 * g, 16)] = full[2 * g]
            srt_i[pl.ds(16 * g, 16)] = full[2 * g + 1]

        zero = jnp.zeros((16,), jnp.float32)
        big = jnp.full((16,), MAX_SPIKE_TIME, jnp.float32)
        one = jnp.ones((16,), jnp.float32)
        carry0 = (zero, zero, zero, zero, zero, zero, zero, zero,
                  big, big, big, big, one, one, one, one)

        def steps(carry, kg, kg1, ig, nsub):
            ws = list(carry[0:4])
            wis = list(carry[4:8])
            bnum = list(carry[8:12])
            bden = list(carry[12:16])
            for sub in range(nsub):
                t = kg[sub]
                nxt = kg1[sub]
                idx = ig[sub]
                for c in range(4):
                    wv = w_v[idx, pl.ds(16 * c, 16)]
                    ws[c] = ws[c] + wv
                    wis[c] = wis[c] + wv * t
                    den = ws[c] - 1.0
                    u = ((wis[c] >= t * den) & (wis[c] <= nxt * den)
                         & (wis[c] * bden[c] < bnum[c] * den))
                    bnum[c] = jnp.where(u, wis[c], bnum[c])
                    bden[c] = jnp.where(u, den, bden[c])
            return tuple(ws) + tuple(wis) + tuple(bnum) + tuple(bden)

        def group(g, carry):
            kg = srt_k[pl.ds(16 * g, 16)]
            kg1 = srt_k[pl.ds(16 * g + 1, 16)]
            ig = srt_i[pl.ds(16 * g, 16)]
            return steps(carry, kg, kg1, ig, 16)

        res = lax.fori_loop(0, 4, group, carry0)
        res = steps(res, srt_k[pl.ds(64, 16)], srt_k[pl.ds(65, 16)],
                    srt_i[pl.ds(64, 16)], 1)
        for c in range(4):
            out_v[r, pl.ds(16 * c, 16)] = res[8 + c] / res[12 + c]
        return 0

    lax.fori_loop(0, rows_per, row_body, 0)
    pltpu.sync_copy(out_v, o_hbm.at[pl.ds(base, rows_per)])


def kernel(layer_in, weight, delay):
    b, in_size = layer_in.shape
    n, out_size = weight.shape
    info = plsc.get_sparse_core_info()
    nc, ns = info.num_cores, info.num_subcores
    nw = nc * ns
    assert b % nw == 0
    rows_per = b // nw
    mesh = plsc.VectorSubcoreMesh(core_axis_name="c", subcore_axis_name="s")
    f = pl.kernel(
        functools.partial(_sc_body, rows_per=rows_per, nc=nc),
        mesh=mesh,
        compiler_params=pltpu.CompilerParams(needs_layout_passes=False),
        out_type=jax.ShapeDtypeStruct((b, out_size), jnp.float32),
        scratch_types=[
            pltpu.VMEM((rows_per, in_size), jnp.float32),
            pltpu.VMEM((n, out_size), jnp.float32),
            pltpu.VMEM((in_size,), jnp.float32),
            pltpu.VMEM((96,), jnp.float32),
            pltpu.VMEM((96,), jnp.int32),
            pltpu.VMEM((rows_per, out_size), jnp.float32),
        ],
    )
    return f(layer_in, weight, delay)


# speedup vs baseline: 8.8949x; 1.0344x over previous
import functools

import jax
import jax.numpy as jnp
from jax import lax
from jax.experimental import pallas as pl
from jax.experimental.pallas import tpu as pltpu
from jax.experimental.pallas import tpu_sc as plsc

MAX_SPIKE_TIME = 100000.0
_INF = float("inf")


def _minmax(ak, av, bk, bv):
    m = ak <= bk
    return (jnp.where(m, ak, bk), jnp.where(m, av, bv),
            jnp.where(m, bk, ak), jnp.where(m, bv, av))


def _rev(x):
    return lax.rev(x, (0,))


def _merge16x2(ak, av, bk, bv):
    lok, lov, hik, hiv = _minmax(ak, av, _rev(bk), _rev(bv))
    return plsc.sort_key_val(lok, lov) + plsc.sort_key_val(hik, hiv)


def _merge32x2(a, b):
    ak0, av0, ak1, av1 = a
    bk0, bv0, bk1, bv1 = b
    l0k, l0v, h0k, h0v = _minmax(ak0, av0, _rev(bk1), _rev(bv1))
    l1k, l1v, h1k, h1v = _minmax(ak1, av1, _rev(bk0), _rev(bv0))
    a0k, a0v, a1k, a1v = _minmax(l0k, l0v, l1k, l1v)
    b0k, b0v, b1k, b1v = _minmax(h0k, h0v, h1k, h1v)
    return (plsc.sort_key_val(a0k, a0v) + plsc.sort_key_val(a1k, a1v),
            plsc.sort_key_val(b0k, b0v) + plsc.sort_key_val(b1k, b1v))


def _merge64x16(c, dk, dv):
    k0, v0, k1, v1, k2, v2, k3, v3 = c
    m3k, m3v, x3k, x3v = _minmax(k3, v3, _rev(dk), _rev(dv))
    l0k, l0v, h0k, h0v = _minmax(k0, v0, k2, v2)
    l1k, l1v, h1k, h1v = _minmax(k1, v1, m3k, m3v)
    a0k, a0v, a1k, a1v = _minmax(l0k, l0v, l1k, l1v)
    b0k, b0v, b1k, b1v = _minmax(h0k, h0v, h1k, h1v)
    return (plsc.sort_key_val(a0k, a0v) + plsc.sort_key_val(a1k, a1v) +
            plsc.sort_key_val(b0k, b0v) + plsc.sort_key_val(b1k, b1v) +
            plsc.sort_key_val(x3k, x3v))


def _sc_body(x_hbm, w_hbm, d_hbm, o_hbm, x_v, w_v, d_v, srt_k, srt_i, out_v,
             *, rows_per, nc):
    wid = lax.axis_index("s") * nc + lax.axis_index("c")
    base = wid * rows_per
    pltpu.sync_copy(x_hbm.at[pl.ds(base, rows_per)], x_v)
    pltpu.sync_copy(w_hbm, w_v)
    pltpu.sync_copy(d_hbm, d_v)

    iota = lax.iota(jnp.int32, 16)
    scale = [jnp.exp(jnp.maximum(d_v[pl.ds(16 * c, 16)], 0.0))
             for c in range(4)]
    ids = [iota + 16 * c for c in range(5)]
    k4 = jnp.where(iota == 0, jnp.float32(1.0), jnp.float32(_INF))

    def sort_row(r, koff):
        keys = [x_v[r, pl.ds(16 * c, 16)] * scale[c] for c in range(4)] + [k4]
        s = [plsc.sort_key_val(k, v) for k, v in zip(keys, ids)]
        a = _merge16x2(*s[0], *s[1])
        b = _merge16x2(*s[2], *s[3])
        c64a, c64b = _merge32x2(a, b)
        full = _merge64x16(c64a + c64b, *s[4])
        for g in range(5):
            srt_k[pl.ds(koff + 16 * g, 16)] = full[2 * g]
            srt_i[pl.ds(koff + 16 * g, 16)] = full[2 * g + 1]

    def row_body(r, _):
        sort_row(2 * r, 0)
        sort_row(2 * r + 1, 96)

        zero = jnp.zeros((16,), jnp.float32)
        big = jnp.full((16,), MAX_SPIKE_TIME, jnp.float32)
        one = jnp.ones((16,), jnp.float32)
        carry0 = (zero,) * 8 + (big,) * 8 + (one,) * 8 + (zero,) * 8

        def steps(carry, kg, kg1, ig, nsub):
            ws = list(carry[0:8])
            bnum = list(carry[8:16])
            bden = list(carry[16:24])
            wis = list(carry[24:32])
            for sub in range(nsub):
                tt = [kg[p][sub] for p in range(2)]
                nn = [kg1[p][sub] for p in range(2)]
                ii = [ig[p][sub] for p in range(2)]
                for c in range(8):
                    p, cc = divmod(c, 4)
                    t, nxt = tt[p], nn[p]
                    wv = w_v[ii[p], pl.ds(16 * cc, 16)]
                    ws[c] = ws[c] + wv
                    wis[c] = wis[c] + wv * t
                    den = ws[c] - 1.0
                    u = ((wis[c] >= t * den) & (wis[c] <= nxt * den)
                         & (wis[c] * bden[c] < bnum[c] * den))
                    bnum[c] = jnp.where(u, wis[c], bnum[c])
                    bden[c] = jnp.where(u, den, bden[c])
            return tuple(ws) + tuple(bnum) + tuple(bden) + tuple(wis)

        def group(g, carry):
            kg = [srt_k[pl.ds(96 * p + 16 * g, 16)] for p in range(2)]
            kg1 = [srt_k[pl.ds(96 * p + 16 * g + 1, 16)] for p in range(2)]
            ig = [srt_i[pl.ds(96 * p + 16 * g, 16)] for p in range(2)]
            return steps(carry, kg, kg1, ig, 16)

        res = lax.fori_loop(0, 4, group, carry0)
        res = steps(res,
                    [srt_k[pl.ds(96 * p + 64, 16)] for p in range(2)],
                    [srt_k[pl.ds(96 * p + 65, 16)] for p in range(2)],
                    [srt_i[pl.ds(96 * p + 64, 16)] for p in range(2)], 1)
        for c in range(8):
            p, cc = divmod(c, 4)
            out_v[2 * r + p, pl.ds(16 * cc, 16)] = res[8 + c] / res[16 + c]
        return 0

    lax.fori_loop(0, rows_per // 2, row_body, 0)
    pltpu.sync_copy(out_v, o_hbm.at[pl.ds(base, rows_per)])


def kernel(layer_in, weight, delay):
    b, in_size = layer_in.shape
    n, out_size = weight.shape
    info = plsc.get_sparse_core_info()
    nc, ns = info.num_cores, info.num_subcores
    nw = nc * ns
    assert b % nw == 0
    rows_per = b // nw
    mesh = plsc.VectorSubcoreMesh(core_axis_name="c", subcore_axis_name="s")
    f = pl.kernel(
        functools.partial(_sc_body, rows_per=rows_per, nc=nc),
        mesh=mesh,
        compiler_params=pltpu.CompilerParams(needs_layout_passes=False),
        out_type=jax.ShapeDtypeStruct((b, out_size), jnp.float32),
        scratch_types=[
            pltpu.VMEM((rows_per, in_size), jnp.float32),
            pltpu.VMEM((n, out_size), jnp.float32),
            pltpu.VMEM((in_size,), jnp.float32),
            pltpu.VMEM((192,), jnp.float32),
            pltpu.VMEM((192,), jnp.int32),
            pltpu.VMEM((rows_per, out_size), jnp.float32),
        ],
    )
    return f(layer_in, weight, delay)


# speedup vs baseline: 10.5085x; 1.1814x over previous
import functools

import jax
import jax.numpy as jnp
from jax import lax
from jax.experimental import pallas as pl
from jax.experimental.pallas import tpu as pltpu
from jax.experimental.pallas import tpu_sc as plsc

MAX_SPIKE_TIME = 100000.0
_INF = float("inf")


def _minmax(ak, av, bk, bv):
    m = ak <= bk
    return (jnp.where(m, ak, bk), jnp.where(m, av, bv),
            jnp.where(m, bk, ak), jnp.where(m, bv, av))


def _rev(x):
    return lax.rev(x, (0,))


def _merge16x2(ak, av, bk, bv):
    lok, lov, hik, hiv = _minmax(ak, av, _rev(bk), _rev(bv))
    return plsc.sort_key_val(lok, lov) + plsc.sort_key_val(hik, hiv)


def _merge32x2(a, b):
    ak0, av0, ak1, av1 = a
    bk0, bv0, bk1, bv1 = b
    l0k, l0v, h0k, h0v = _minmax(ak0, av0, _rev(bk1), _rev(bv1))
    l1k, l1v, h1k, h1v = _minmax(ak1, av1, _rev(bk0), _rev(bv0))
    a0k, a0v, a1k, a1v = _minmax(l0k, l0v, l1k, l1v)
    b0k, b0v, b1k, b1v = _minmax(h0k, h0v, h1k, h1v)
    return (plsc.sort_key_val(a0k, a0v) + plsc.sort_key_val(a1k, a1v),
            plsc.sort_key_val(b0k, b0v) + plsc.sort_key_val(b1k, b1v))


def _merge64x16(c, dk, dv):
    k0, v0, k1, v1, k2, v2, k3, v3 = c
    m3k, m3v, x3k, x3v = _minmax(k3, v3, _rev(dk), _rev(dv))
    l0k, l0v, h0k, h0v = _minmax(k0, v0, k2, v2)
    l1k, l1v, h1k, h1v = _minmax(k1, v1, m3k, m3v)
    a0k, a0v, a1k, a1v = _minmax(l0k, l0v, l1k, l1v)
    b0k, b0v, b1k, b1v = _minmax(h0k, h0v, h1k, h1v)
    return (plsc.sort_key_val(a0k, a0v) + plsc.sort_key_val(a1k, a1v) +
            plsc.sort_key_val(b0k, b0v) + plsc.sort_key_val(b1k, b1v) +
            plsc.sort_key_val(x3k, x3v))


def _sc_body(x_hbm, w_hbm, d_hbm, o_hbm, x_v, w_v, d_v, srt_k, srt_i, out_v,
             *, rows_per, nc):
    wid = lax.axis_index("s") * nc + lax.axis_index("c")
    base = wid * rows_per
    pltpu.sync_copy(x_hbm.at[pl.ds(base, rows_per)], x_v)
    pltpu.sync_copy(w_hbm, w_v)
    pltpu.sync_copy(d_hbm, d_v)

    iota = lax.iota(jnp.int32, 16)
    scale = [jnp.exp(jnp.maximum(d_v[pl.ds(16 * c, 16)], 0.0))
             for c in range(4)]
    ids = [iota + 16 * c for c in range(5)]
    k4 = jnp.where(iota == 0, jnp.float32(1.0), jnp.float32(_INF))

    def sort_row(r, koff):
        keys = [x_v[r, pl.ds(16 * c, 16)] * scale[c] for c in range(4)] + [k4]
        s = [plsc.sort_key_val(k, v) for k, v in zip(keys, ids)]
        a = _merge16x2(*s[0], *s[1])
        b = _merge16x2(*s[2], *s[3])
        c64a, c64b = _merge32x2(a, b)
        full = _merge64x16(c64a + c64b, *s[4])
        for g in range(5):
            srt_k[pl.ds(koff + 16 * g, 16)] = full[2 * g]
            srt_i[pl.ds(koff + 16 * g, 16)] = full[2 * g + 1]

    def row_body(r, _):
        sort_row(2 * r, 0)
        sort_row(2 * r + 1, 96)

        zero = jnp.zeros((16,), jnp.float32)
        big = jnp.full((16,), MAX_SPIKE_TIME, jnp.float32)
        one = jnp.ones((16,), jnp.float32)
        carry0 = (zero,) * 8 + (big,) * 8 + (one,) * 8 + (zero,) * 8

        def steps(carry, kg, kg1, ig, nsub):
            ws = list(carry[0:8])
            bnum = list(carry[8:16])
            bden = list(carry[16:24])
            wis = list(carry[24:32])
            for sub in range(nsub):
                tt = [kg[p][sub] for p in range(2)]
                nn = [kg1[p][sub] for p in range(2)]
                ii = [ig[p][sub] for p in range(2)]
                for c in range(8):
                    p, cc = divmod(c, 4)
                    t, nxt = tt[p], nn[p]
                    wv = w_v[ii[p], pl.ds(16 * cc, 16)]
                    ws[c] = ws[c] + wv
                    wis[c] = wis[c] + wv * t
                    den = ws[c] - 1.0
                    u = ((wis[c] >= t * den) & (wis[c] <= nxt * den)
                         & (wis[c] * bden[c] < bnum[c] * den))
                    bnum[c] = jnp.where(u, wis[c], bnum[c])
                    bden[c] = jnp.where(u, den, bden[c])
            return tuple(ws) + tuple(bnum) + tuple(bden) + tuple(wis)

        def group(g, carry):
            kg = [srt_k[pl.ds(96 * p + 16 * g, 16)] for p in range(2)]
            kg1 = [srt_k[pl.ds(96 * p + 16 * g + 1, 16)] for p in range(2)]
            ig = [srt_i[pl.ds(96 * p + 16 * g, 16)] for p in range(2)]
            return steps(carry, kg, kg1, ig, 16)

        res = lax.fori_loop(0, 4, group, carry0)
        res = steps(res,
                    [srt_k[pl.ds(96 * p + 64, 16)] for p in range(2)],
                    [srt_k[pl.ds(96 * p + 65, 16)] for p in range(2)],
                    [srt_i[pl.ds(96 * p + 64, 16)] for p in range(2)], 1)
        for c in range(8):
            p, cc = divmod(c, 4)
            out_v[2 * r + p, pl.ds(16 * cc, 16)] = res[8 + c] / res[16 + c]
        return 0

    lax.fori_loop(0, rows_per // 2, row_body, 0)
    pltpu.sync_copy(out_v, o_hbm.at[pl.ds(base, rows_per)])


def _sc_kernel_entry(layer_in, weight, delay):
    b, in_size = layer_in.shape
    n, out_size = weight.shape
    info = plsc.get_sparse_core_info()
    nc, ns = info.num_cores, info.num_subcores
    nw = nc * ns
    assert b % nw == 0
    rows_per = b // nw
    mesh = plsc.VectorSubcoreMesh(core_axis_name="c", subcore_axis_name="s")
    f = pl.kernel(
        functools.partial(_sc_body, rows_per=rows_per, nc=nc),
        mesh=mesh,
        compiler_params=pltpu.CompilerParams(needs_layout_passes=False),
        out_type=jax.ShapeDtypeStruct((b, out_size), jnp.float32),
        scratch_types=[
            pltpu.VMEM((rows_per, in_size), jnp.float32),
            pltpu.VMEM((n, out_size), jnp.float32),
            pltpu.VMEM((in_size,), jnp.float32),
            pltpu.VMEM((192,), jnp.float32),
            pltpu.VMEM((192,), jnp.int32),
            pltpu.VMEM((rows_per, out_size), jnp.float32),
        ],
    )
    return f(layer_in, weight, delay)


def _tc_body(x_ref, w_ref, d_ref, o_ref, *, n):
    x = x_ref[...]
    d = d_ref[...]
    bb = x.shape[0]
    x = x * jnp.exp(jnp.maximum(d, 0.0))
    xs = jnp.concatenate([x, jnp.ones((bb, 1), jnp.float32)], axis=1)
    w = w_ref[...]
    lane = jax.lax.broadcasted_iota(jnp.int32, (1, n), 1)

    def body(k, acc):
        t = jnp.max(jnp.where(lane == k, xs, -jnp.inf), axis=1, keepdims=True)
        le = (xs < t) | ((xs == t) & (lane <= k))
        m = le.astype(jnp.float32)
        dn = (((1,), (0,)), ((), ()))
        sw = jax.lax.dot_general(m, w, dn, precision=jax.lax.Precision.HIGHEST,
                                 preferred_element_type=jnp.float32)
        swx = jax.lax.dot_general(m * xs, w, dn,
                                  precision=jax.lax.Precision.HIGHEST,
                                  preferred_element_type=jnp.float32)
        nxt = jnp.min(jnp.where(le, MAX_SPIKE_TIME, xs), axis=1, keepdims=True)
        den = jnp.maximum(sw - 1.0, 1e-10)
        cand = swx / den
        cand = jnp.where(sw < 1.0, MAX_SPIKE_TIME, cand)
        cand = jnp.where(cand < t, MAX_SPIKE_TIME, cand)
        cand = jnp.where(cand > nxt, MAX_SPIKE_TIME, cand)
        return jnp.minimum(acc, cand)

    acc0 = jnp.full((bb, w.shape[1]), MAX_SPIKE_TIME, jnp.float32)
    o_ref[...] = jax.lax.fori_loop(0, n, body, acc0)


def _tc_kernel(layer_in, weight, delay, block_b):
    b, in_size = layer_in.shape
    out_size = weight.shape[1]
    n = in_size + 1
    bb = block_b if b % block_b == 0 else b
    d2 = delay.reshape(1, in_size)
    return pl.pallas_call(
        functools.partial(_tc_body, n=n),
        grid=(b // bb,),
        in_specs=[
            pl.BlockSpec((bb, in_size), lambda i: (i, 0)),
            pl.BlockSpec((n, out_size), lambda i: (0, 0)),
            pl.BlockSpec((1, in_size), lambda i: (0, 0)),
        ],
        out_specs=pl.BlockSpec((bb, out_size), lambda i: (i, 0)),
        out_shape=jax.ShapeDtypeStruct((b, out_size), jnp.float32),
    )(layer_in, weight, d2)


N_SC = 3072


def kernel(layer_in, weight, delay):
    out_sc = _sc_kernel_entry(layer_in[:N_SC], weight, delay)
    out_tc = _tc_kernel(layer_in[N_SC:], weight, delay, block_b=512)
    return jnp.concatenate([out_sc, out_tc], axis=0)
